# baseline (speedup 1.0000x reference)
import functools

import jax
import jax.numpy as jnp
from jax import lax
from jax.experimental import pallas as pl
from jax.experimental.pallas import tpu as pltpu
from jax.experimental.pallas import tpu_sc as plsc

_EPS = 1e-5
_NGRAPH = 256
_NC, _NS = 2, 16
_NW = _NC * _NS
_B = 128
_EBLK = 2560

_f32 = jnp.float32


def _embed_body(x_ref, w_ref, b_ref, o_ref):
    o_ref[...] = jnp.dot(x_ref[...], w_ref[...],
                         preferred_element_type=_f32) + b_ref[...]


def _embed(x, W, b):
    N = x.shape[0]
    d = W.shape[1]
    return pl.pallas_call(
        _embed_body,
        out_shape=jax.ShapeDtypeStruct((N, d), _f32),
    )(x, W, b.reshape(1, -1))


def _edge_t(xi, xj, ea, wi, wj, we, b):
    t = jnp.dot(xi[...], wi[...], preferred_element_type=_f32)
    t = t + jnp.dot(xj[...], wj[...], preferred_element_type=_f32)
    t = t + jnp.dot(ea[...], we[...], preferred_element_type=_f32)
    return t + b[...]


def _stats_body(xi, xj, ea, wi, wj, we, b, st_ref):
    t = _edge_t(xi, xj, ea, wi, wj, we, b)
    s = jnp.sum(t, axis=0, keepdims=True)
    ss = jnp.sum(t * t, axis=0, keepdims=True)
    upd = jnp.concatenate(
        [s, ss, jnp.zeros((6, s.shape[1]), _f32)], axis=0)

    @pl.when(pl.program_id(0) == 0)
    def _():
        st_ref[...] = upd

    @pl.when(pl.program_id(0) != 0)
    def _():
        st_ref[...] = st_ref[...] + upd


def _msg_body(xi, xj, ea, wi, wj, we, b, st, g1, be1, o_ref, *, n_edges):
    t = _edge_t(xi, xj, ea, wi, wj, we, b)
    mu = st[0:1, :] / n_edges
    var = st[1:2, :] / n_edges - mu * mu
    alpha = g1[...] * lax.rsqrt(var + _EPS)
    beta = be1[...] - mu * alpha
    tn = t * alpha + beta
    half = tn.shape[1] // 2
    filt = jax.nn.sigmoid(tn[:, :half])
    core = jax.nn.softplus(tn[:, half:])
    o_ref[...] = filt * core


def _edge_specs(E, d, nbr):
    nblk = E // _EBLK
    in_specs = [
        pl.BlockSpec((_EBLK, d), lambda i: (i, 0)),
        pl.BlockSpec((_EBLK, d), lambda i: (i, 0)),
        pl.BlockSpec((_EBLK, nbr), lambda i: (i, 0)),
        pl.BlockSpec((d, 2 * d), lambda i: (0, 0)),
        pl.BlockSpec((d, 2 * d), lambda i: (0, 0)),
        pl.BlockSpec((nbr, 2 * d), lambda i: (0, 0)),
        pl.BlockSpec((1, 2 * d), lambda i: (0, 0)),
    ]
    return nblk, in_specs


def _edge_stats(E, xi, xj, ea, wi, wj, we, b):
    d = xi.shape[1]
    nbr = ea.shape[1]
    nblk, in_specs = _edge_specs(E, d, nbr)
    return pl.pallas_call(
        _stats_body,
        grid=(nblk,),
        in_specs=in_specs,
        out_specs=pl.BlockSpec((8, 2 * d), lambda i: (0, 0)),
        out_shape=jax.ShapeDtypeStruct((8, 2 * d), _f32),
    )(xi, xj, ea, wi, wj, we, b)


def _edge_msg(E, xi, xj, ea, wi, wj, we, b, st, g1, be1):
    Epad, d = xi.shape
    nbr = ea.shape[1]
    nblk, in_specs = _edge_specs(E, d, nbr)
    in_specs += [
        pl.BlockSpec((8, 2 * d), lambda i: (0, 0)),
        pl.BlockSpec((1, 2 * d), lambda i: (0, 0)),
        pl.BlockSpec((1, 2 * d), lambda i: (0, 0)),
    ]
    body = functools.partial(_msg_body, n_edges=float(E))
    return pl.pallas_call(
        body,
        grid=(nblk,),
        in_specs=in_specs,
        out_specs=pl.BlockSpec((_EBLK, d), lambda i: (i, 0)),
        out_shape=jax.ShapeDtypeStruct((Epad, d), _f32),
    )(xi, xj, ea, wi, wj, we, b, st, g1.reshape(1, -1), be1.reshape(1, -1))


def _update_body(p0, p1, h, g2, be2, o_ref):
    aggr = p0[...] + p1[...]
    n = aggr.shape[0]
    mu = jnp.sum(aggr, axis=0, keepdims=True) / n
    var = jnp.sum(aggr * aggr, axis=0, keepdims=True) / n - mu * mu
    an = g2[...] * (aggr - mu) * lax.rsqrt(var + _EPS) + be2[...]
    o_ref[...] = jax.nn.softplus(h[...] + an)


def _update(p0, p1, h, g2, be2):
    N, d = h.shape
    spec = pl.BlockSpec((N, d), lambda i: (0, 0))
    vspec = pl.BlockSpec((1, d), lambda i: (0, 0))
    return pl.pallas_call(
        _update_body,
        grid=(1,),
        in_specs=[spec, spec, spec, vspec, vspec],
        out_specs=spec,
        out_shape=jax.ShapeDtypeStruct((N, d), _f32),
    )(p0, p1, h, g2.reshape(1, -1), be2.reshape(1, -1))


def _head_body(s0, s1, c0, c1, wfc, bfc, wout, bout, o_ref):
    sums = s0[...] + s1[...]
    cnts = jnp.maximum(c0[...] + c1[...], 1.0)
    crys = jax.nn.softplus(sums / cnts)
    crys = jax.nn.softplus(
        jnp.dot(crys, wfc[...], preferred_element_type=_f32) + bfc[...])
    o_ref[...] = jnp.dot(crys, wout[...],
                         preferred_element_type=_f32) + bout[...]


def _head(s0, s1, c0, c1, wfc, bfc, wout_pad, bout_pad):
    ng = _NGRAPH
    d = wfc.shape[0]
    hw = wfc.shape[1]
    gspec = pl.BlockSpec((ng, d), lambda i: (0, 0))
    return pl.pallas_call(
        _head_body,
        grid=(1,),
        in_specs=[gspec, gspec, gspec, gspec,
                  pl.BlockSpec((d, hw), lambda i: (0, 0)),
                  pl.BlockSpec((1, hw), lambda i: (0, 0)),
                  pl.BlockSpec((hw, hw), lambda i: (0, 0)),
                  pl.BlockSpec((1, hw), lambda i: (0, 0))],
        out_specs=pl.BlockSpec((ng, hw), lambda i: (0, 0)),
        out_shape=jax.ShapeDtypeStruct((ng, hw), _f32),
    )(s0, s1, c0, c1, wfc, bfc, wout_pad, bout_pad)


def _sc_mesh():
    return plsc.VectorSubcoreMesh(core_axis_name="c", subcore_axis_name="s",
                                  num_cores=_NC, num_subcores=_NS)


def _make_gather(N, Epad, D):
    per_w = Epad // _NW
    nb = per_w // _B

    G = next(g for g in (5, 4, 2, 1) if nb % g == 0)
    total_b = nb * _NW
    nb0 = total_b // _NS
    nb1 = total_b // _NS - nb0

    def body(h_hbm, dsti_hbm, srci_hbm, xi_hbm, xj_hbm,
             idxd, idxs, bufa, bufb, sema, semb, semw):
        c = lax.axis_index("c")
        s = lax.axis_index("s")

        def run(nbw, rb):
            eb = rb * _B
            pltpu.sync_copy(dsti_hbm.at[pl.ds(rb, nbw)],
                            idxd.at[pl.ds(0, nbw)])
            pltpu.sync_copy(srci_hbm.at[pl.ds(rb, nbw)],
                            idxs.at[pl.ds(0, nbw)])

            def group(g, carry):
                da, db = [], []
                for b in range(G):
                    j = g * G + b
                    sl = pl.ds(b * _B, _B)
                    da.append(pltpu.async_copy(
                        h_hbm.at[idxd.at[j]], bufa.at[sl], sema))
                    db.append(pltpu.async_copy(
                        h_hbm.at[idxs.at[j]], bufb.at[sl], semb))
                for b in range(G):
                    da[b].wait()
                    db[b].wait()
                dst = pl.ds(eb + g * G * _B, G * _B)
                wa = pltpu.async_copy(bufa, xi_hbm.at[dst], semw)
                wb = pltpu.async_copy(bufb, xj_hbm.at[dst], semw)
                wa.wait()
                wb.wait()
                return carry

            lax.fori_loop(0, nbw // G, group, 0)

        @pl.when(c == 0)
        def _():
            run(nb0, s * nb0)

        if nb1 > 0:
            @pl.when(c == 1)
            def _():
                run(nb1, _NS * nb0 + s * nb1)

    return pl.kernel(
        body,
        out_type=[jax.ShapeDtypeStruct((Epad, D), _f32),
                  jax.ShapeDtypeStruct((Epad, D), _f32)],
        mesh=_sc_mesh(),
        compiler_params=pltpu.CompilerParams(use_tc_tiling_on_sc=False),
        scratch_types=[
            pltpu.VMEM((max(nb0, nb1), _B), jnp.int32),
            pltpu.VMEM((max(nb0, nb1), _B), jnp.int32),
            pltpu.VMEM((G * _B, D), _f32),
            pltpu.VMEM((G * _B, D), _f32),
            pltpu.SemaphoreType.DMA,
            pltpu.SemaphoreType.DMA,
            pltpu.SemaphoreType.DMA,
        ],
    )


def _make_scatter(R, T, D, with_counts, full_idx):
    per_w = R // _NW
    nb = per_w // _B
    rpt = T // _NS
    G = next(g for g in (5, 4, 3, 2, 1) if nb % g == 0)

    n_out = 4 if with_counts else 2

    def body(*refs):
        if with_counts:
            (rows_hbm, idx_hbm, zero_hbm, ones_hbm,
             s0_hbm, s1_hbm, c0_hbm, c1_hbm,
             idxv, rbuf, obuf, shared, shared_cnt, semr) = refs
        else:
            (rows_hbm, idx_hbm, zero_hbm,
             s0_hbm, s1_hbm,
             idxv, rbuf, shared, semr) = refs
        c = lax.axis_index("c")
        s = lax.axis_index("s")
        wid = s * _NC + c

        stage = rbuf.at[pl.ds(0, rpt)]
        pltpu.sync_copy(zero_hbm.at[pl.ds(s * rpt, rpt)], stage)
        pltpu.sync_copy(stage, shared.at[pl.ds(s * rpt, rpt)])
        if with_counts:
            pltpu.sync_copy(stage, shared_cnt.at[pl.ds(s * rpt, rpt)])
            pltpu.sync_copy(ones_hbm, obuf)
        plsc.subcore_barrier()

        if full_idx:
            pltpu.sync_copy(idx_hbm, idxv)
        else:
            pltpu.sync_copy(idx_hbm.at[pl.ds(wid * nb, nb)], idxv)

        def group(g, carry):
            ld = []
            for b in range(G):
                j = g * G + b
                ld.append(pltpu.async_copy(
                    rows_hbm.at[pl.ds(wid * per_w + j * _B, _B)],
                    rbuf.at[pl.ds(b * _B, _B)], semr))
            for b in range(G):
                j = g * G + b
                ld[b].wait()
                row = idxv.at[wid * nb + j] if full_idx else idxv.at[j]
                pltpu.sync_copy(rbuf.at[pl.ds(b * _B, _B)],
                                shared.at[row], add=True)
                if with_counts:
                    pltpu.sync_copy(obuf, shared_cnt.at[row], add=True)
            return carry

        lax.fori_loop(0, nb // G, group, 0)
        plsc.subcore_barrier()

        @pl.when(c == 0)
        def _():
            pltpu.sync_copy(shared.at[pl.ds(s * rpt, rpt)], stage)
            pltpu.sync_copy(stage, s0_hbm.at[pl.ds(s * rpt, rpt)])
            if with_counts:
                pltpu.sync_copy(shared_cnt.at[pl.ds(s * rpt, rpt)], stage)
                pltpu.sync_copy(stage, c0_hbm.at[pl.ds(s * rpt, rpt)])

        @pl.when(c == 1)
        def _():
            pltpu.sync_copy(shared.at[pl.ds(s * rpt, rpt)], stage)
            pltpu.sync_copy(stage, s1_hbm.at[pl.ds(s * rpt, rpt)])
            if with_counts:
                pltpu.sync_copy(shared_cnt.at[pl.ds(s * rpt, rpt)], stage)
                pltpu.sync_copy(stage, c1_hbm.at[pl.ds(s * rpt, rpt)])

    idx_rows = (R // _B) if full_idx else nb
    scratch = [
        pltpu.VMEM((idx_rows, _B), jnp.int32),
        pltpu.VMEM((max(G * _B, rpt), D), _f32),
    ]
    if with_counts:
        scratch.append(pltpu.VMEM((_B, D), _f32))
    scratch.append(pltpu.VMEM_SHARED((T, D), _f32))
    if with_counts:
        scratch.append(pltpu.VMEM_SHARED((T, D), _f32))
    scratch.append(pltpu.SemaphoreType.DMA)

    return pl.kernel(
        body,
        out_type=[jax.ShapeDtypeStruct((T, D), _f32)] * n_out,
        mesh=_sc_mesh(),
        compiler_params=pltpu.CompilerParams(use_tc_tiling_on_sc=False),
        scratch_types=scratch,
    )


def kernel(x, edge_index, edge_attr, batch, W_emb, b_emb, W_full, b_full,
           g1, be1, g2, be2, W_fc, b_fc, W_out, b_out):
    N = x.shape[0]
    E = edge_index.shape[1]
    d = W_emb.shape[1]
    n_conv = W_full.shape[0]

    grain = _NW * _B
    egrain = grain * 8
    Epad = ((E + egrain - 1) // egrain) * egrain
    Tn = ((N + _B - 1) // _B) * _B
    if Tn == N:
        Tn = N + _B

    src = edge_index[0]
    dst = edge_index[1]
    epad = Epad - E
    srcg = jnp.concatenate([src, jnp.zeros((epad,), jnp.int32)])
    dstg = jnp.concatenate([dst, jnp.zeros((epad,), jnp.int32)])
    dsts = jnp.concatenate([dst, jnp.full((epad,), N, jnp.int32)])
    srcg = srcg.reshape(Epad // _B, _B)
    dstg = dstg.reshape(Epad // _B, _B)
    dsts = dsts.reshape(Epad // _B, _B)

    h = _embed(x, W_emb, b_emb)

    gather = _make_gather(N, Epad, d)
    scat_msg = _make_scatter(Epad, Tn, d, with_counts=False, full_idx=False)
    zeros_T = jnp.zeros((Tn, d), _f32)

    for l in range(n_conv):
        xi, xj = gather(h, dstg, srcg)
        wi = W_full[l, :d]
        wj = W_full[l, d:2 * d]
        we = W_full[l, 2 * d:]
        bias = b_full[l].reshape(1, -1)
        st = _edge_stats(E, xi, xj, edge_attr, wi, wj, we, bias)
        msg = _edge_msg(E, xi, xj, edge_attr, wi, wj, we, bias, st,
                        g1[l], be1[l])
        p0, p1 = scat_msg(msg, dsts, zeros_T)
        h = _update(p0, p1, h, g2[l], be2[l])

    NP = ((N + grain - 1) // grain) * grain
    Tg = ((_NGRAPH // _NS) + 8) * _NS
    h_pad = jnp.concatenate([h, jnp.zeros((NP - N, d), _f32)], axis=0)
    b_pad = jnp.concatenate(
        [batch, jnp.full((NP - N,), _NGRAPH, jnp.int32)]).reshape(
            NP // _B, _B)
    scat_pool = _make_scatter(NP, Tg, d, with_counts=True, full_idx=True)
    s0, s1, c0, c1 = scat_pool(h_pad, b_pad, jnp.zeros((Tg, d), _f32),
                               jnp.ones((_B, d), _f32))

    hdim = W_fc.shape[1]
    wout_pad = jnp.concatenate(
        [W_out, jnp.zeros((hdim, hdim - W_out.shape[1]), _f32)], axis=1)
    bout_pad = jnp.concatenate(
        [b_out, jnp.zeros((hdim - b_out.shape[0],), _f32)]).reshape(1, -1)
    out_full = _head(s0, s1, c0, c1,
                     W_fc, b_fc.reshape(1, -1), wout_pad, bout_pad)
    return out_full[:, :W_out.shape[1]]


# speedup vs baseline: 1.4630x; 1.4630x over previous
import functools

import jax
import jax.numpy as jnp
from jax import lax
from jax.experimental import pallas as pl
from jax.experimental.pallas import tpu as pltpu
from jax.experimental.pallas import tpu_sc as plsc

_EPS = 1e-5
_NGRAPH = 256
_NC, _NS = 2, 16
_NW = _NC * _NS
_B = 128
_EBLK = 2560

_f32 = jnp.float32


def _embed_body(x_ref, w_ref, b_ref, o_ref):
    o_ref[...] = jnp.dot(x_ref[...], w_ref[...],
                         preferred_element_type=_f32) + b_ref[...]


def _embed(x, W, b):
    N = x.shape[0]
    d = W.shape[1]
    return pl.pallas_call(
        _embed_body,
        out_shape=jax.ShapeDtypeStruct((N, d), _f32),
    )(x, W, b.reshape(1, -1))


def _edge_t(xi, xj, ea, wi, wj, we, b):
    t = jnp.dot(xi[...], wi[...], preferred_element_type=_f32)
    t = t + jnp.dot(xj[...], wj[...], preferred_element_type=_f32)
    t = t + jnp.dot(ea[...], we[...], preferred_element_type=_f32)
    return t + b[...]


def _stats_body(xi, xj, ea, wi, wj, we, b, st_ref):
    t = _edge_t(xi, xj, ea, wi, wj, we, b)
    s2 = jnp.sum(t, axis=0, keepdims=True)
    ss2 = jnp.sum(t * t, axis=0, keepdims=True)
    hw = s2.shape[1] // 2
    s = s2[:, :hw] + s2[:, hw:]
    ss = ss2[:, :hw] + ss2[:, hw:]
    upd = jnp.concatenate(
        [s, ss, jnp.zeros((6, hw), _f32)], axis=0)

    @pl.when(pl.program_id(0) == 0)
    def _():
        st_ref[...] = upd

    @pl.when(pl.program_id(0) != 0)
    def _():
        st_ref[...] = st_ref[...] + upd


def _msg_body(xi, xj, ea, wi, wj, we, b, st, g1, be1, o_ref, *, n_edges):
    t = _edge_t(xi, xj, ea, wi, wj, we, b)
    mu = st[0:1, :] / n_edges
    var = st[1:2, :] / n_edges - mu * mu
    alpha = g1[...] * lax.rsqrt(var + _EPS)
    beta = be1[...] - mu * alpha
    a2 = jnp.concatenate([alpha, alpha], axis=1)
    b2 = jnp.concatenate([beta, beta], axis=1)
    tn = t * a2 + b2
    q = tn.shape[1] // 4
    f0 = jax.nn.sigmoid(tn[:, :q])
    c0 = jax.nn.softplus(tn[:, q:2 * q])
    f1 = jax.nn.sigmoid(tn[:, 2 * q:3 * q])
    c1 = jax.nn.softplus(tn[:, 3 * q:])
    o_ref[...] = jnp.concatenate([f0 * c0, f1 * c1], axis=1)


def _edge_specs(nblk, d, nbr):
    in_specs = [
        pl.BlockSpec((_EBLK // 2, 2 * d), lambda i: (i, 0)),
        pl.BlockSpec((_EBLK // 2, 2 * d), lambda i: (i, 0)),
        pl.BlockSpec((_EBLK // 2, 2 * nbr), lambda i: (i, 0)),
        pl.BlockSpec((2 * d, 4 * d), lambda i: (0, 0)),
        pl.BlockSpec((2 * d, 4 * d), lambda i: (0, 0)),
        pl.BlockSpec((2 * nbr, 4 * d), lambda i: (0, 0)),
        pl.BlockSpec((1, 4 * d), lambda i: (0, 0)),
    ]
    return in_specs


def _edge_stats(E, d, nbr, xi2, xj2, ea2, wi2, wj2, we2, b2):
    nblk = E // _EBLK
    in_specs = _edge_specs(nblk, d, nbr)
    return pl.pallas_call(
        _stats_body,
        grid=(nblk,),
        in_specs=in_specs,
        out_specs=pl.BlockSpec((8, 2 * d), lambda i: (0, 0)),
        out_shape=jax.ShapeDtypeStruct((8, 2 * d), _f32),
    )(xi2, xj2, ea2, wi2, wj2, we2, b2)


def _edge_msg(E, d, nbr, xi2, xj2, ea2, wi2, wj2, we2, b2, st, g1, be1):
    nblk = E // _EBLK
    Ep2 = xi2.shape[0]
    in_specs = _edge_specs(nblk, d, nbr)
    in_specs += [
        pl.BlockSpec((8, 2 * d), lambda i: (0, 0)),
        pl.BlockSpec((1, 2 * d), lambda i: (0, 0)),
        pl.BlockSpec((1, 2 * d), lambda i: (0, 0)),
    ]
    body = functools.partial(_msg_body, n_edges=float(E))
    return pl.pallas_call(
        body,
        grid=(nblk,),
        in_specs=in_specs,
        out_specs=pl.BlockSpec((_EBLK // 2, 2 * d), lambda i: (i, 0)),
        out_shape=jax.ShapeDtypeStruct((Ep2, 2 * d), _f32),
    )(xi2, xj2, ea2, wi2, wj2, we2, b2, st,
      g1.reshape(1, -1), be1.reshape(1, -1))


def _update_body(p0, p1, h, g2, be2, o_ref):
    aggr = p0[...] + p1[...]
    n = aggr.shape[0]
    mu = jnp.sum(aggr, axis=0, keepdims=True) / n
    var = jnp.sum(aggr * aggr, axis=0, keepdims=True) / n - mu * mu
    an = g2[...] * (aggr - mu) * lax.rsqrt(var + _EPS) + be2[...]
    o_ref[...] = jax.nn.softplus(h[...] + an)


def _update(p0, p1, h, g2, be2):
    N, d = h.shape
    spec = pl.BlockSpec((N, d), lambda i: (0, 0))
    vspec = pl.BlockSpec((1, d), lambda i: (0, 0))
    return pl.pallas_call(
        _update_body,
        grid=(1,),
        in_specs=[spec, spec, spec, vspec, vspec],
        out_specs=spec,
        out_shape=jax.ShapeDtypeStruct((N, d), _f32),
    )(p0, p1, h, g2.reshape(1, -1), be2.reshape(1, -1))


def _head_body(s0, s1, c0, c1, wfc, bfc, wout, bout, o_ref):
    sums = s0[...] + s1[...]
    cnts = jnp.maximum(c0[...] + c1[...], 1.0)
    crys = jax.nn.softplus(sums / cnts)
    crys = jax.nn.softplus(
        jnp.dot(crys, wfc[...], preferred_element_type=_f32) + bfc[...])
    o_ref[...] = jnp.dot(crys, wout[...],
                         preferred_element_type=_f32) + bout[...]


def _head(s0, s1, c0, c1, wfc, bfc, wout_pad, bout_pad):
    ng = _NGRAPH
    d = wfc.shape[0]
    hw = wfc.shape[1]
    gspec = pl.BlockSpec((ng, d), lambda i: (0, 0))
    return pl.pallas_call(
        _head_body,
        grid=(1,),
        in_specs=[gspec, gspec, gspec, gspec,
                  pl.BlockSpec((d, hw), lambda i: (0, 0)),
                  pl.BlockSpec((1, hw), lambda i: (0, 0)),
                  pl.BlockSpec((hw, hw), lambda i: (0, 0)),
                  pl.BlockSpec((1, hw), lambda i: (0, 0))],
        out_specs=pl.BlockSpec((ng, hw), lambda i: (0, 0)),
        out_shape=jax.ShapeDtypeStruct((ng, hw), _f32),
    )(s0, s1, c0, c1, wfc, bfc, wout_pad, bout_pad)


def _sc_mesh():
    return plsc.VectorSubcoreMesh(core_axis_name="c", subcore_axis_name="s",
                                  num_cores=_NC, num_subcores=_NS)


def _make_gather(N, Epad, D):
    per_w = Epad // _NW
    nb = per_w // _B

    G = next(g for g in (5, 4, 2, 1) if nb % g == 0)
    total_b = nb * _NW
    nb0 = total_b // (2 * _NS)
    nb1 = total_b // _NS - nb0

    def body(h_hbm, dsti_hbm, srci_hbm, xi_hbm, xj_hbm,
             idxd, idxs, bufa, bufb, sema, semb, semw):
        c = lax.axis_index("c")
        s = lax.axis_index("s")

        def run(nbw, rb):
            eb = rb * _B
            pltpu.sync_copy(dsti_hbm.at[pl.ds(rb, nbw)],
                            idxd.at[pl.ds(0, nbw)])
            pltpu.sync_copy(srci_hbm.at[pl.ds(rb, nbw)],
                            idxs.at[pl.ds(0, nbw)])

            def group(g, carry):
                da, db = [], []
                for b in range(G):
                    j = g * G + b
                    sl = pl.ds(b * _B, _B)
                    da.append(pltpu.async_copy(
                        h_hbm.at[idxd.at[j]], bufa.at[sl], sema))
                    db.append(pltpu.async_copy(
                        h_hbm.at[idxs.at[j]], bufb.at[sl], semb))
                for b in range(G):
                    da[b].wait()
                    db[b].wait()
                dst = pl.ds(eb + g * G * _B, G * _B)
                wa = pltpu.async_copy(bufa, xi_hbm.at[dst], semw)
                wb = pltpu.async_copy(bufb, xj_hbm.at[dst], semw)
                wa.wait()
                wb.wait()
                return carry

            lax.fori_loop(0, nbw // G, group, 0)

        @pl.when(c == 0)
        def _():
            run(nb0, s * nb0)

        if nb1 > 0:
            @pl.when(c == 1)
            def _():
                run(nb1, _NS * nb0 + s * nb1)

    return pl.kernel(
        body,
        out_type=[jax.ShapeDtypeStruct((Epad, D), _f32),
                  jax.ShapeDtypeStruct((Epad, D), _f32)],
        mesh=_sc_mesh(),
        compiler_params=pltpu.CompilerParams(use_tc_tiling_on_sc=False),
        scratch_types=[
            pltpu.VMEM((max(nb0, nb1), _B), jnp.int32),
            pltpu.VMEM((max(nb0, nb1), _B), jnp.int32),
            pltpu.VMEM((G * _B, D), _f32),
            pltpu.VMEM((G * _B, D), _f32),
            pltpu.SemaphoreType.DMA,
            pltpu.SemaphoreType.DMA,
            pltpu.SemaphoreType.DMA,
        ],
    )


def _make_scatter(R, T, D, with_counts, full_idx):
    per_w = R // _NW
    nb = per_w // _B
    rpt = T // _NS
    G = next(g for g in (5, 4, 3, 2, 1) if nb % g == 0)

    n_out = 4 if with_counts else 2

    def body(*refs):
        if with_counts:
            (rows_hbm, idx_hbm, zero_hbm, ones_hbm,
             s0_hbm, s1_hbm, c0_hbm, c1_hbm,
             idxv, rbuf, obuf, shared, shared_cnt, semr) = refs
        else:
            (rows_hbm, idx_hbm, zero_hbm,
             s0_hbm, s1_hbm,
             idxv, rbuf, shared, semr) = refs
        c = lax.axis_index("c")
        s = lax.axis_index("s")
        wid = s * _NC + c

        stage = rbuf.at[pl.ds(0, rpt)]
        pltpu.sync_copy(zero_hbm.at[pl.ds(s * rpt, rpt)], stage)
        pltpu.sync_copy(stage, shared.at[pl.ds(s * rpt, rpt)])
        if with_counts:
            pltpu.sync_copy(stage, shared_cnt.at[pl.ds(s * rpt, rpt)])
            pltpu.sync_copy(ones_hbm, obuf)
        plsc.subcore_barrier()

        if full_idx:
            pltpu.sync_copy(idx_hbm, idxv)
        else:
            pltpu.sync_copy(idx_hbm.at[pl.ds(wid * nb, nb)], idxv)

        def group(g, carry):
            ld = []
            for b in range(G):
                j = g * G + b
                ld.append(pltpu.async_copy(
                    rows_hbm.at[pl.ds(wid * per_w + j * _B, _B)],
                    rbuf.at[pl.ds(b * _B, _B)], semr))
            for b in range(G):
                j = g * G + b
                ld[b].wait()
                row = idxv.at[wid * nb + j] if full_idx else idxv.at[j]
                pltpu.sync_copy(rbuf.at[pl.ds(b * _B, _B)],
                                shared.at[row], add=True)
                if with_counts:
                    pltpu.sync_copy(obuf, shared_cnt.at[row], add=True)
            return carry

        lax.fori_loop(0, nb // G, group, 0)
        plsc.subcore_barrier()

        @pl.when(c == 0)
        def _():
            pltpu.sync_copy(shared.at[pl.ds(s * rpt, rpt)], stage)
            pltpu.sync_copy(stage, s0_hbm.at[pl.ds(s * rpt, rpt)])
            if with_counts:
                pltpu.sync_copy(shared_cnt.at[pl.ds(s * rpt, rpt)], stage)
                pltpu.sync_copy(stage, c0_hbm.at[pl.ds(s * rpt, rpt)])

        @pl.when(c == 1)
        def _():
            pltpu.sync_copy(shared.at[pl.ds(s * rpt, rpt)], stage)
            pltpu.sync_copy(stage, s1_hbm.at[pl.ds(s * rpt, rpt)])
            if with_counts:
                pltpu.sync_copy(shared_cnt.at[pl.ds(s * rpt, rpt)], stage)
                pltpu.sync_copy(stage, c1_hbm.at[pl.ds(s * rpt, rpt)])

    idx_rows = (R // _B) if full_idx else nb
    scratch = [
        pltpu.VMEM((idx_rows, _B), jnp.int32),
        pltpu.VMEM((max(G * _B, rpt), D), _f32),
    ]
    if with_counts:
        scratch.append(pltpu.VMEM((_B, D), _f32))
    scratch.append(pltpu.VMEM_SHARED((T, D), _f32))
    if with_counts:
        scratch.append(pltpu.VMEM_SHARED((T, D), _f32))
    scratch.append(pltpu.SemaphoreType.DMA)

    return pl.kernel(
        body,
        out_type=[jax.ShapeDtypeStruct((T, D), _f32)] * n_out,
        mesh=_sc_mesh(),
        compiler_params=pltpu.CompilerParams(use_tc_tiling_on_sc=False),
        scratch_types=scratch,
    )


def kernel(x, edge_index, edge_attr, batch, W_emb, b_emb, W_full, b_full,
           g1, be1, g2, be2, W_fc, b_fc, W_out, b_out):
    N = x.shape[0]
    E = edge_index.shape[1]
    d = W_emb.shape[1]
    n_conv = W_full.shape[0]

    grain = _NW * _B
    egrain = grain * 8
    Epad = ((E + egrain - 1) // egrain) * egrain
    Tn = ((N + _B - 1) // _B) * _B
    if Tn == N:
        Tn = N + _B

    src = edge_index[0]
    dst = edge_index[1]
    epad = Epad - E
    srcg = jnp.concatenate([src, jnp.zeros((epad,), jnp.int32)])
    dstg = jnp.concatenate([dst, jnp.zeros((epad,), jnp.int32)])
    dsts = jnp.concatenate([dst, jnp.full((epad,), N, jnp.int32)])
    srcg = srcg.reshape(Epad // _B, _B)
    dstg = dstg.reshape(Epad // _B, _B)
    dsts = dsts.reshape(Epad // _B, _B)

    h = _embed(x, W_emb, b_emb)

    gather = _make_gather(N, Epad, d)
    scat_msg = _make_scatter(Epad, Tn, d, with_counts=False, full_idx=False)
    zeros_T = jnp.zeros((Tn, d), _f32)

    nbr = edge_attr.shape[1]
    ea2 = jnp.concatenate(
        [edge_attr, jnp.zeros((epad, nbr), _f32)]).reshape(Epad // 2, 2 * nbr)

    def blockdiag(w):
        z = jnp.zeros_like(w)
        return jnp.concatenate(
            [jnp.concatenate([w, z], axis=1),
             jnp.concatenate([z, w], axis=1)], axis=0)

    for l in range(n_conv):
        xi, xj = gather(h, dstg, srcg)
        xi2 = xi.reshape(Epad // 2, 2 * d)
        xj2 = xj.reshape(Epad // 2, 2 * d)
        wi2 = blockdiag(W_full[l, :d])
        wj2 = blockdiag(W_full[l, d:2 * d])
        we2 = blockdiag(W_full[l, 2 * d:])
        bias2 = jnp.concatenate([b_full[l], b_full[l]]).reshape(1, -1)
        st = _edge_stats(E, d, nbr, xi2, xj2, ea2, wi2, wj2, we2, bias2)
        msg2 = _edge_msg(E, d, nbr, xi2, xj2, ea2, wi2, wj2, we2, bias2, st,
                         g1[l], be1[l])
        msg = msg2.reshape(Epad, d)
        p0, p1 = scat_msg(msg, dsts, zeros_T)
        h = _update(p0, p1, h, g2[l], be2[l])

    NP = ((N + grain - 1) // grain) * grain
    Tg = ((_NGRAPH // _NS) + 8) * _NS
    h_pad = jnp.concatenate([h, jnp.zeros((NP - N, d), _f32)], axis=0)
    b_pad = jnp.concatenate(
        [batch, jnp.full((NP - N,), _NGRAPH, jnp.int32)]).reshape(
            NP // _B, _B)
    scat_pool = _make_scatter(NP, Tg, d, with_counts=True, full_idx=True)
    s0, s1, c0, c1 = scat_pool(h_pad, b_pad, jnp.zeros((Tg, d), _f32),
                               jnp.ones((_B, d), _f32))

    hdim = W_fc.shape[1]
    wout_pad = jnp.concatenate(
        [W_out, jnp.zeros((hdim, hdim - W_out.shape[1]), _f32)], axis=1)
    bout_pad = jnp.concatenate(
        [b_out, jnp.zeros((hdim - b_out.shape[0],), _f32)]).reshape(1, -1)
    out_full = _head(s0, s1, c0, c1,
                     W_fc, b_fc.reshape(1, -1), wout_pad, bout_pad)
    return out_full[:, :W_out.shape[1]]


# speedup vs baseline: 1.5370x; 1.0506x over previous
import functools

import jax
import jax.numpy as jnp
from jax import lax
from jax.experimental import pallas as pl
from jax.experimental.pallas import tpu as pltpu
from jax.experimental.pallas import tpu_sc as plsc

_EPS = 1e-5
_NGRAPH = 256
_NC, _NS = 2, 16
_NW = _NC * _NS
_B = 128
_EBLK = 2560

_f32 = jnp.float32


def _embed_body(x_ref, w_ref, b_ref, o_ref):
    o_ref[...] = jnp.dot(x_ref[...], w_ref[...],
                         preferred_element_type=_f32) + b_ref[...]


def _embed(x, W, b):
    N = x.shape[0]
    d = W.shape[1]
    return pl.pallas_call(
        _embed_body,
        out_shape=jax.ShapeDtypeStruct((N, d), _f32),
    )(x, W, b.reshape(1, -1))


def _edge_t(xi, xj, ea, wi, wj, we, b):
    t = jnp.dot(xi[...], wi[...], preferred_element_type=_f32)
    t = t + jnp.dot(xj[...], wj[...], preferred_element_type=_f32)
    t = t + jnp.dot(ea[...], we[...], preferred_element_type=_f32)
    return t + b[...]


def _stats_body(xi, xj, ea, wi, wj, we, b, st_ref):
    t = _edge_t(xi, xj, ea, wi, wj, we, b)
    s2 = jnp.sum(t, axis=0, keepdims=True)
    ss2 = jnp.sum(t * t, axis=0, keepdims=True)
    hw = s2.shape[1] // 2
    s = s2[:, :hw] + s2[:, hw:]
    ss = ss2[:, :hw] + ss2[:, hw:]
    upd = jnp.concatenate(
        [s, ss, jnp.zeros((6, hw), _f32)], axis=0)

    @pl.when(pl.program_id(0) == 0)
    def _():
        st_ref[...] = upd

    @pl.when(pl.program_id(0) != 0)
    def _():
        st_ref[...] = st_ref[...] + upd


def _msg_body(xi, xj, ea, wi, wj, we, b, sta, stb, g1, be1, o_ref, *,
              n_edges):
    t = _edge_t(xi, xj, ea, wi, wj, we, b)
    mu = (sta[0:1, :] + stb[0:1, :]) / n_edges
    var = (sta[1:2, :] + stb[1:2, :]) / n_edges - mu * mu
    alpha = g1[...] * lax.rsqrt(var + _EPS)
    beta = be1[...] - mu * alpha
    a2 = jnp.concatenate([alpha, alpha], axis=1)
    b2 = jnp.concatenate([beta, beta], axis=1)
    tn = t * a2 + b2
    q = tn.shape[1] // 4
    f0 = jax.nn.sigmoid(tn[:, :q])
    c0 = jax.nn.softplus(tn[:, q:2 * q])
    f1 = jax.nn.sigmoid(tn[:, 2 * q:3 * q])
    c1 = jax.nn.softplus(tn[:, 3 * q:])
    o_ref[...] = jnp.concatenate([f0 * c0, f1 * c1], axis=1)


def _edge_specs(nblk, d, nbr):
    in_specs = [
        pl.BlockSpec((_EBLK // 2, 2 * d), lambda i: (i, 0)),
        pl.BlockSpec((_EBLK // 2, 2 * d), lambda i: (i, 0)),
        pl.BlockSpec((_EBLK // 2, 2 * nbr), lambda i: (i, 0)),
        pl.BlockSpec((2 * d, 4 * d), lambda i: (0, 0)),
        pl.BlockSpec((2 * d, 4 * d), lambda i: (0, 0)),
        pl.BlockSpec((2 * nbr, 4 * d), lambda i: (0, 0)),
        pl.BlockSpec((1, 4 * d), lambda i: (0, 0)),
    ]
    return in_specs


def _edge_stats(E, d, nbr, xi2, xj2, ea2, wi2, wj2, we2, b2):
    nblk = E // _EBLK
    in_specs = _edge_specs(nblk, d, nbr)
    return pl.pallas_call(
        _stats_body,
        grid=(nblk,),
        in_specs=in_specs,
        out_specs=pl.BlockSpec((8, 2 * d), lambda i: (0, 0)),
        out_shape=jax.ShapeDtypeStruct((8, 2 * d), _f32),
    )(xi2, xj2, ea2, wi2, wj2, we2, b2)


def _edge_msg(E, Etot, d, nbr, xi2, xj2, ea2, wi2, wj2, we2, b2,
              sta, stb, g1, be1):
    nblk = E // _EBLK
    Ep2 = xi2.shape[0]
    in_specs = _edge_specs(nblk, d, nbr)
    in_specs += [
        pl.BlockSpec((8, 2 * d), lambda i: (0, 0)),
        pl.BlockSpec((8, 2 * d), lambda i: (0, 0)),
        pl.BlockSpec((1, 2 * d), lambda i: (0, 0)),
        pl.BlockSpec((1, 2 * d), lambda i: (0, 0)),
    ]
    body = functools.partial(_msg_body, n_edges=float(Etot))
    return pl.pallas_call(
        body,
        grid=(nblk,),
        in_specs=in_specs,
        out_specs=pl.BlockSpec((_EBLK // 2, 2 * d), lambda i: (i, 0)),
        out_shape=jax.ShapeDtypeStruct((Ep2, 2 * d), _f32),
    )(xi2, xj2, ea2, wi2, wj2, we2, b2, sta, stb,
      g1.reshape(1, -1), be1.reshape(1, -1))


def _update_body(p0, p1, p2, p3, h, g2, be2, o_ref):
    aggr = (p0[...] + p1[...]) + (p2[...] + p3[...])
    n = aggr.shape[0]
    mu = jnp.sum(aggr, axis=0, keepdims=True) / n
    var = jnp.sum(aggr * aggr, axis=0, keepdims=True) / n - mu * mu
    an = g2[...] * (aggr - mu) * lax.rsqrt(var + _EPS) + be2[...]
    o_ref[...] = jax.nn.softplus(h[...] + an)


def _update(p0, p1, p2, p3, h, g2, be2):
    N, d = h.shape
    spec = pl.BlockSpec((N, d), lambda i: (0, 0))
    vspec = pl.BlockSpec((1, d), lambda i: (0, 0))
    return pl.pallas_call(
        _update_body,
        grid=(1,),
        in_specs=[spec, spec, spec, spec, spec, vspec, vspec],
        out_specs=spec,
        out_shape=jax.ShapeDtypeStruct((N, d), _f32),
    )(p0, p1, p2, p3, h, g2.reshape(1, -1), be2.reshape(1, -1))


def _head_body(s0, s1, c0, c1, wfc, bfc, wout, bout, o_ref):
    sums = s0[...] + s1[...]
    cnts = jnp.maximum(c0[...] + c1[...], 1.0)
    crys = jax.nn.softplus(sums / cnts)
    crys = jax.nn.softplus(
        jnp.dot(crys, wfc[...], preferred_element_type=_f32) + bfc[...])
    o_ref[...] = jnp.dot(crys, wout[...],
                         preferred_element_type=_f32) + bout[...]


def _head(s0, s1, c0, c1, wfc, bfc, wout_pad, bout_pad):
    ng = _NGRAPH
    d = wfc.shape[0]
    hw = wfc.shape[1]
    gspec = pl.BlockSpec((ng, d), lambda i: (0, 0))
    return pl.pallas_call(
        _head_body,
        grid=(1,),
        in_specs=[gspec, gspec, gspec, gspec,
                  pl.BlockSpec((d, hw), lambda i: (0, 0)),
                  pl.BlockSpec((1, hw), lambda i: (0, 0)),
                  pl.BlockSpec((hw, hw), lambda i: (0, 0)),
                  pl.BlockSpec((1, hw), lambda i: (0, 0))],
        out_specs=pl.BlockSpec((ng, hw), lambda i: (0, 0)),
        out_shape=jax.ShapeDtypeStruct((ng, hw), _f32),
    )(s0, s1, c0, c1, wfc, bfc, wout_pad, bout_pad)


def _sc_mesh():
    return plsc.VectorSubcoreMesh(core_axis_name="c", subcore_axis_name="s",
                                  num_cores=_NC, num_subcores=_NS)


def _make_gather(N, Epad, D):
    per_w = Epad // _NW
    nb = per_w // _B

    G = next(g for g in (5, 4, 2, 1) if nb % g == 0)
    total_b = nb * _NW
    nb0 = total_b // (2 * _NS)
    nb1 = total_b // _NS - nb0

    def body(h_hbm, dsti_hbm, srci_hbm, xi_hbm, xj_hbm,
             idxd, idxs, bufa, bufb, sema, semb, semw):
        c = lax.axis_index("c")
        s = lax.axis_index("s")

        def run(nbw, rb):
            eb = rb * _B
            pltpu.sync_copy(dsti_hbm.at[pl.ds(rb, nbw)],
                            idxd.at[pl.ds(0, nbw)])
            pltpu.sync_copy(srci_hbm.at[pl.ds(rb, nbw)],
                            idxs.at[pl.ds(0, nbw)])

            def group(g, carry):
                da, db = [], []
                for b in range(G):
                    j = g * G + b
                    sl = pl.ds(b * _B, _B)
                    da.append(pltpu.async_copy(
                        h_hbm.at[idxd.at[j]], bufa.at[sl], sema))
                    db.append(pltpu.async_copy(
                        h_hbm.at[idxs.at[j]], bufb.at[sl], semb))
                for b in range(G):
                    da[b].wait()
                    db[b].wait()
                dst = pl.ds(eb + g * G * _B, G * _B)
                wa = pltpu.async_copy(bufa, xi_hbm.at[dst], semw)
                wb = pltpu.async_copy(bufb, xj_hbm.at[dst], semw)
                wa.wait()
                wb.wait()
                return carry

            lax.fori_loop(0, nbw // G, group, 0)

        @pl.when(c == 0)
        def _():
            run(nb0, s * nb0)

        if nb1 > 0:
            @pl.when(c == 1)
            def _():
                run(nb1, _NS * nb0 + s * nb1)

    return pl.kernel(
        body,
        out_type=[jax.ShapeDtypeStruct((Epad, D), _f32),
                  jax.ShapeDtypeStruct((Epad, D), _f32)],
        mesh=_sc_mesh(),
        compiler_params=pltpu.CompilerParams(use_tc_tiling_on_sc=False),
        scratch_types=[
            pltpu.VMEM((max(nb0, nb1), _B), jnp.int32),
            pltpu.VMEM((max(nb0, nb1), _B), jnp.int32),
            pltpu.VMEM((G * _B, D), _f32),
            pltpu.VMEM((G * _B, D), _f32),
            pltpu.SemaphoreType.DMA,
            pltpu.SemaphoreType.DMA,
            pltpu.SemaphoreType.DMA,
        ],
    )


def _make_scatter(R, T, D, with_counts, full_idx):
    per_w = R // _NW
    nb = per_w // _B
    rpt = T // _NS
    G = next(g for g in (5, 4, 3, 2, 1) if nb % g == 0)

    n_out = 4 if with_counts else 2

    def body(*refs):
        if with_counts:
            (rows_hbm, idx_hbm, zero_hbm, ones_hbm,
             s0_hbm, s1_hbm, c0_hbm, c1_hbm,
             idxv, rbuf, obuf, shared, shared_cnt, semr) = refs
        else:
            (rows_hbm, idx_hbm, zero_hbm,
             s0_hbm, s1_hbm,
             idxv, rbuf, shared, semr) = refs
        c = lax.axis_index("c")
        s = lax.axis_index("s")
        wid = s * _NC + c

        stage = rbuf.at[pl.ds(0, rpt)]
        pltpu.sync_copy(zero_hbm.at[pl.ds(s * rpt, rpt)], stage)
        pltpu.sync_copy(stage, shared.at[pl.ds(s * rpt, rpt)])
        if with_counts:
            pltpu.sync_copy(stage, shared_cnt.at[pl.ds(s * rpt, rpt)])
            pltpu.sync_copy(ones_hbm, obuf)
        plsc.subcore_barrier()

        if full_idx:
            pltpu.sync_copy(idx_hbm, idxv)
        else:
            pltpu.sync_copy(idx_hbm.at[pl.ds(wid * nb, nb)], idxv)

        def group(g, carry):
            ld = []
            for b in range(G):
                j = g * G + b
                ld.append(pltpu.async_copy(
                    rows_hbm.at[pl.ds(wid * per_w + j * _B, _B)],
                    rbuf.at[pl.ds(b * _B, _B)], semr))
            for b in range(G):
                j = g * G + b
                ld[b].wait()
                row = idxv.at[wid * nb + j] if full_idx else idxv.at[j]
                pltpu.sync_copy(rbuf.at[pl.ds(b * _B, _B)],
                                shared.at[row], add=True)
                if with_counts:
                    pltpu.sync_copy(obuf, shared_cnt.at[row], add=True)
            return carry

        lax.fori_loop(0, nb // G, group, 0)
        plsc.subcore_barrier()

        @pl.when(c == 0)
        def _():
            pltpu.sync_copy(shared.at[pl.ds(s * rpt, rpt)], stage)
            pltpu.sync_copy(stage, s0_hbm.at[pl.ds(s * rpt, rpt)])
            if with_counts:
                pltpu.sync_copy(shared_cnt.at[pl.ds(s * rpt, rpt)], stage)
                pltpu.sync_copy(stage, c0_hbm.at[pl.ds(s * rpt, rpt)])

        @pl.when(c == 1)
        def _():
            pltpu.sync_copy(shared.at[pl.ds(s * rpt, rpt)], stage)
            pltpu.sync_copy(stage, s1_hbm.at[pl.ds(s * rpt, rpt)])
            if with_counts:
                pltpu.sync_copy(shared_cnt.at[pl.ds(s * rpt, rpt)], stage)
                pltpu.sync_copy(stage, c1_hbm.at[pl.ds(s * rpt, rpt)])

    idx_rows = (R // _B) if full_idx else nb
    scratch = [
        pltpu.VMEM((idx_rows, _B), jnp.int32),
        pltpu.VMEM((max(G * _B, rpt), D), _f32),
    ]
    if with_counts:
        scratch.append(pltpu.VMEM((_B, D), _f32))
    scratch.append(pltpu.VMEM_SHARED((T, D), _f32))
    if with_counts:
        scratch.append(pltpu.VMEM_SHARED((T, D), _f32))
    scratch.append(pltpu.SemaphoreType.DMA)

    return pl.kernel(
        body,
        out_type=[jax.ShapeDtypeStruct((T, D), _f32)] * n_out,
        mesh=_sc_mesh(),
        compiler_params=pltpu.CompilerParams(use_tc_tiling_on_sc=False),
        scratch_types=scratch,
    )


def kernel(x, edge_index, edge_attr, batch, W_emb, b_emb, W_full, b_full,
           g1, be1, g2, be2, W_fc, b_fc, W_out, b_out):
    N = x.shape[0]
    E = edge_index.shape[1]
    d = W_emb.shape[1]
    n_conv = W_full.shape[0]

    grain = _NW * _B
    egrain = grain * 8
    Epad = ((E + egrain - 1) // egrain) * egrain
    Tn = ((N + _B - 1) // _B) * _B
    if Tn == N:
        Tn = N + _B

    src = edge_index[0]
    dst = edge_index[1]
    epad = Epad - E
    srcg = jnp.concatenate([src, jnp.zeros((epad,), jnp.int32)])
    dstg = jnp.concatenate([dst, jnp.zeros((epad,), jnp.int32)])
    dsts = jnp.concatenate([dst, jnp.full((epad,), N, jnp.int32)])
    srcg = srcg.reshape(Epad // _B, _B)
    dstg = dstg.reshape(Epad // _B, _B)
    dsts = dsts.reshape(Epad // _B, _B)

    h = _embed(x, W_emb, b_emb)

    Eh = Epad // 2
    rh = Eh // _B
    realE = (min(E, Eh), E - min(E, Eh))
    gather = _make_gather(N, Eh, d)
    scat_msg = _make_scatter(Eh, Tn, d, with_counts=False, full_idx=False)
    zeros_T = jnp.zeros((Tn, d), _f32)

    nbr = edge_attr.shape[1]
    ea_pad = jnp.concatenate([edge_attr, jnp.zeros((epad, nbr), _f32)])
    ea2_h = [ea_pad[i * Eh:(i + 1) * Eh].reshape(Eh // 2, 2 * nbr)
             for i in range(2)]
    dstg_h = [dstg[i * rh:(i + 1) * rh] for i in range(2)]
    srcg_h = [srcg[i * rh:(i + 1) * rh] for i in range(2)]
    dsts_h = [dsts[i * rh:(i + 1) * rh] for i in range(2)]

    def blockdiag(w):
        z = jnp.zeros_like(w)
        return jnp.concatenate(
            [jnp.concatenate([w, z], axis=1),
             jnp.concatenate([z, w], axis=1)], axis=0)

    for l in range(n_conv):
        wi2 = blockdiag(W_full[l, :d])
        wj2 = blockdiag(W_full[l, d:2 * d])
        we2 = blockdiag(W_full[l, 2 * d:])
        bias2 = jnp.concatenate([b_full[l], b_full[l]]).reshape(1, -1)
        xs, sts, msgs = [], [], []
        for i in range(2):
            xi, xj = gather(h, dstg_h[i], srcg_h[i])
            xs.append((xi.reshape(Eh // 2, 2 * d),
                       xj.reshape(Eh // 2, 2 * d)))
            sts.append(_edge_stats(realE[i], d, nbr, xs[i][0], xs[i][1],
                                   ea2_h[i], wi2, wj2, we2, bias2))
        parts = []
        for i in range(2):
            msg2 = _edge_msg(realE[i], E, d, nbr, xs[i][0], xs[i][1],
                             ea2_h[i], wi2, wj2, we2, bias2,
                             sts[0], sts[1], g1[l], be1[l])
            p0, p1 = scat_msg(msg2.reshape(Eh, d), dsts_h[i], zeros_T)
            parts += [p0, p1]
        h = _update(parts[0], parts[1], parts[2], parts[3],
                    h, g2[l], be2[l])

    NP = ((N + grain - 1) // grain) * grain
    Tg = ((_NGRAPH // _NS) + 8) * _NS
    h_pad = jnp.concatenate([h, jnp.zeros((NP - N, d), _f32)], axis=0)
    b_pad = jnp.concatenate(
        [batch, jnp.full((NP - N,), _NGRAPH, jnp.int32)]).reshape(
            NP // _B, _B)
    scat_pool = _make_scatter(NP, Tg, d, with_counts=True, full_idx=True)
    s0, s1, c0, c1 = scat_pool(h_pad, b_pad, jnp.zeros((Tg, d), _f32),
                               jnp.ones((_B, d), _f32))

    hdim = W_fc.shape[1]
    wout_pad = jnp.concatenate(
        [W_out, jnp.zeros((hdim, hdim - W_out.shape[1]), _f32)], axis=1)
    bout_pad = jnp.concatenate(
        [b_out, jnp.zeros((hdim - b_out.shape[0],), _f32)]).reshape(1, -1)
    out_full = _head(s0, s1, c0, c1,
                     W_fc, b_fc.reshape(1, -1), wout_pad, bout_pad)
    return out_full[:, :W_out.shape[1]]


# speedup vs baseline: 1.6057x; 1.0447x over previous
import functools

import jax
import jax.numpy as jnp
from jax import lax
from jax.experimental import pallas as pl
from jax.experimental.pallas import tpu as pltpu
from jax.experimental.pallas import tpu_sc as plsc

_EPS = 1e-5
_NGRAPH = 256
_NC, _NS = 2, 16
_NW = _NC * _NS
_B = 128
_EBLK = 2560

_f32 = jnp.float32


def _embed_body(x_ref, w_ref, b_ref, o_ref):
    o_ref[...] = jnp.dot(x_ref[...], w_ref[...],
                         preferred_element_type=_f32) + b_ref[...]


def _embed(x, W, b):
    N = x.shape[0]
    d = W.shape[1]
    return pl.pallas_call(
        _embed_body,
        out_shape=jax.ShapeDtypeStruct((N, d), _f32),
    )(x, W, b.reshape(1, -1))


def _edge_t(xi, xj, ea, wi, wj, we, b):
    t = jnp.dot(xi[...], wi[...], preferred_element_type=_f32)
    t = t + jnp.dot(xj[...], wj[...], preferred_element_type=_f32)
    t = t + jnp.dot(ea[...], we[...], preferred_element_type=_f32)
    return t + b[...]


def _stats_body(xi, xj, ea, wi, wj, we, b, st_ref):
    t = _edge_t(xi, xj, ea, wi, wj, we, b)
    s2 = jnp.sum(t, axis=0, keepdims=True)
    ss2 = jnp.sum(t * t, axis=0, keepdims=True)
    hw = s2.shape[1] // 2
    s = s2[:, :hw] + s2[:, hw:]
    ss = ss2[:, :hw] + ss2[:, hw:]
    upd = jnp.concatenate(
        [s, ss, jnp.zeros((6, hw), _f32)], axis=0)

    @pl.when(pl.program_id(0) == 0)
    def _():
        st_ref[...] = upd

    @pl.when(pl.program_id(0) != 0)
    def _():
        st_ref[...] = st_ref[...] + upd


def _msg_body(xi, xj, ea, wi, wj, we, b, st, g1, be1, o_ref, *,
              n_edges):
    t = _edge_t(xi, xj, ea, wi, wj, we, b)
    mu = st[0:1, :] / n_edges
    var = st[1:2, :] / n_edges - mu * mu
    alpha = g1[...] * lax.rsqrt(var + _EPS)
    beta = be1[...] - mu * alpha
    a2 = jnp.concatenate([alpha, alpha], axis=1)
    b2 = jnp.concatenate([beta, beta], axis=1)
    tn = t * a2 + b2
    q = tn.shape[1] // 4
    f0 = jax.nn.sigmoid(tn[:, :q])
    c0 = jax.nn.softplus(tn[:, q:2 * q])
    f1 = jax.nn.sigmoid(tn[:, 2 * q:3 * q])
    c1 = jax.nn.softplus(tn[:, 3 * q:])
    o_ref[...] = jnp.concatenate([f0 * c0, f1 * c1], axis=1)


def _edge_specs(nblk, d, nbr):
    in_specs = [
        pl.BlockSpec((_EBLK // 2, 2 * d), lambda i: (i, 0)),
        pl.BlockSpec((_EBLK // 2, 2 * d), lambda i: (i, 0)),
        pl.BlockSpec((_EBLK // 2, 2 * nbr), lambda i: (i, 0)),
        pl.BlockSpec((2 * d, 4 * d), lambda i: (0, 0)),
        pl.BlockSpec((2 * d, 4 * d), lambda i: (0, 0)),
        pl.BlockSpec((2 * nbr, 4 * d), lambda i: (0, 0)),
        pl.BlockSpec((1, 4 * d), lambda i: (0, 0)),
    ]
    return in_specs


def _edge_stats(E, d, nbr, xi2, xj2, ea2, wi2, wj2, we2, b2):
    nblk = E // _EBLK
    in_specs = _edge_specs(nblk, d, nbr)
    return pl.pallas_call(
        _stats_body,
        grid=(nblk,),
        in_specs=in_specs,
        out_specs=pl.BlockSpec((8, 2 * d), lambda i: (0, 0)),
        out_shape=jax.ShapeDtypeStruct((8, 2 * d), _f32),
    )(xi2, xj2, ea2, wi2, wj2, we2, b2)


def _edge_msg(E, Etot, d, nbr, xi2, xj2, ea2, wi2, wj2, we2, b2,
              st, g1, be1):
    nblk = E // _EBLK
    Ep2 = xi2.shape[0]
    in_specs = _edge_specs(nblk, d, nbr)
    in_specs += [
        pl.BlockSpec((8, 2 * d), lambda i: (0, 0)),
        pl.BlockSpec((1, 2 * d), lambda i: (0, 0)),
        pl.BlockSpec((1, 2 * d), lambda i: (0, 0)),
    ]
    body = functools.partial(_msg_body, n_edges=float(Etot))
    return pl.pallas_call(
        body,
        grid=(nblk,),
        in_specs=in_specs,
        out_specs=pl.BlockSpec((_EBLK // 2, 2 * d), lambda i: (i, 0)),
        out_shape=jax.ShapeDtypeStruct((Ep2, 2 * d), _f32),
    )(xi2, xj2, ea2, wi2, wj2, we2, b2, st,
      g1.reshape(1, -1), be1.reshape(1, -1))


def _update_body(*refs):
    parts, (h, g2, be2, o_ref) = refs[:-4], refs[-4:]
    aggr = parts[0][...]
    for p in parts[1:]:
        aggr = aggr + p[...]
    n = aggr.shape[0]
    mu = jnp.sum(aggr, axis=0, keepdims=True) / n
    var = jnp.sum(aggr * aggr, axis=0, keepdims=True) / n - mu * mu
    an = g2[...] * (aggr - mu) * lax.rsqrt(var + _EPS) + be2[...]
    o_ref[...] = jax.nn.softplus(h[...] + an)


def _update(parts, h, g2, be2):
    N, d = h.shape
    spec = pl.BlockSpec((N, d), lambda i: (0, 0))
    vspec = pl.BlockSpec((1, d), lambda i: (0, 0))
    return pl.pallas_call(
        _update_body,
        grid=(1,),
        in_specs=[spec] * (len(parts) + 1) + [vspec, vspec],
        out_specs=spec,
        out_shape=jax.ShapeDtypeStruct((N, d), _f32),
    )(*parts, h, g2.reshape(1, -1), be2.reshape(1, -1))


def _head_body(s0, s1, c0, c1, wfc, bfc, wout, bout, o_ref):
    sums = s0[...] + s1[...]
    cnts = jnp.maximum(c0[...] + c1[...], 1.0)
    crys = jax.nn.softplus(sums / cnts)
    crys = jax.nn.softplus(
        jnp.dot(crys, wfc[...], preferred_element_type=_f32) + bfc[...])
    o_ref[...] = jnp.dot(crys, wout[...],
                         preferred_element_type=_f32) + bout[...]


def _head(s0, s1, c0, c1, wfc, bfc, wout_pad, bout_pad):
    ng = _NGRAPH
    d = wfc.shape[0]
    hw = wfc.shape[1]
    gspec = pl.BlockSpec((ng, d), lambda i: (0, 0))
    return pl.pallas_call(
        _head_body,
        grid=(1,),
        in_specs=[gspec, gspec, gspec, gspec,
                  pl.BlockSpec((d, hw), lambda i: (0, 0)),
                  pl.BlockSpec((1, hw), lambda i: (0, 0)),
                  pl.BlockSpec((hw, hw), lambda i: (0, 0)),
                  pl.BlockSpec((1, hw), lambda i: (0, 0))],
        out_specs=pl.BlockSpec((ng, hw), lambda i: (0, 0)),
        out_shape=jax.ShapeDtypeStruct((ng, hw), _f32),
    )(s0, s1, c0, c1, wfc, bfc, wout_pad, bout_pad)


def _sc_mesh():
    return plsc.VectorSubcoreMesh(core_axis_name="c", subcore_axis_name="s",
                                  num_cores=_NC, num_subcores=_NS)


def _make_gather(N, Epad, D):
    per_w = Epad // _NW
    nb = per_w // _B

    G = next(g for g in (5, 4, 2, 1) if nb % g == 0)
    total_b = nb * _NW
    nb0 = total_b // (2 * _NS)
    nb1 = total_b // _NS - nb0

    def body(h_hbm, dsti_hbm, srci_hbm, xi_hbm, xj_hbm,
             idxd, idxs, bufa, bufb, sema, semb, semw):
        c = lax.axis_index("c")
        s = lax.axis_index("s")

        def run(nbw, rb):
            eb = rb * _B
            pltpu.sync_copy(dsti_hbm.at[pl.ds(rb, nbw)],
                            idxd.at[pl.ds(0, nbw)])
            pltpu.sync_copy(srci_hbm.at[pl.ds(rb, nbw)],
                            idxs.at[pl.ds(0, nbw)])

            def group(g, carry):
                da, db = [], []
                for b in range(G):
                    j = g * G + b
                    sl = pl.ds(b * _B, _B)
                    da.append(pltpu.async_copy(
                        h_hbm.at[idxd.at[j]], bufa.at[sl], sema))
                    db.append(pltpu.async_copy(
                        h_hbm.at[idxs.at[j]], bufb.at[sl], semb))
                for b in range(G):
                    da[b].wait()
                    db[b].wait()
                dst = pl.ds(eb + g * G * _B, G * _B)
                wa = pltpu.async_copy(bufa, xi_hbm.at[dst], semw)
                wb = pltpu.async_copy(bufb, xj_hbm.at[dst], semw)
                wa.wait()
                wb.wait()
                return carry

            lax.fori_loop(0, nbw // G, group, 0)

        @pl.when(c == 0)
        def _():
            run(nb0, s * nb0)

        if nb1 > 0:
            @pl.when(c == 1)
            def _():
                run(nb1, _NS * nb0 + s * nb1)

    return pl.kernel(
        body,
        out_type=[jax.ShapeDtypeStruct((Epad, D), _f32),
                  jax.ShapeDtypeStruct((Epad, D), _f32)],
        mesh=_sc_mesh(),
        compiler_params=pltpu.CompilerParams(use_tc_tiling_on_sc=False),
        scratch_types=[
            pltpu.VMEM((max(nb0, nb1), _B), jnp.int32),
            pltpu.VMEM((max(nb0, nb1), _B), jnp.int32),
            pltpu.VMEM((G * _B, D), _f32),
            pltpu.VMEM((G * _B, D), _f32),
            pltpu.SemaphoreType.DMA,
            pltpu.SemaphoreType.DMA,
            pltpu.SemaphoreType.DMA,
        ],
    )


def _make_scatter(R, T, D, with_counts, full_idx):
    per_w = R // _NW
    nb = per_w // _B
    rpt = T // _NS
    G = next(g for g in (5, 4, 3, 2, 1) if nb % g == 0)

    n_out = 4 if with_counts else 2

    def body(*refs):
        if with_counts:
            (rows_hbm, idx_hbm, zero_hbm, ones_hbm,
             s0_hbm, s1_hbm, c0_hbm, c1_hbm,
             idxv, rbuf, obuf, shared, shared_cnt, semr) = refs
        else:
            (rows_hbm, idx_hbm, zero_hbm,
             s0_hbm, s1_hbm,
             idxv, rbuf, shared, semr) = refs
        c = lax.axis_index("c")
        s = lax.axis_index("s")
        wid = s * _NC + c

        stage = rbuf.at[pl.ds(0, rpt)]
        pltpu.sync_copy(zero_hbm.at[pl.ds(s * rpt, rpt)], stage)
        pltpu.sync_copy(stage, shared.at[pl.ds(s * rpt, rpt)])
        if with_counts:
            pltpu.sync_copy(stage, shared_cnt.at[pl.ds(s * rpt, rpt)])
            pltpu.sync_copy(ones_hbm, obuf)
        plsc.subcore_barrier()

        if full_idx:
            pltpu.sync_copy(idx_hbm, idxv)
        else:
            pltpu.sync_copy(idx_hbm.at[pl.ds(wid * nb, nb)], idxv)

        def group(g, carry):
            ld = []
            for b in range(G):
                j = g * G + b
                ld.append(pltpu.async_copy(
                    rows_hbm.at[pl.ds(wid * per_w + j * _B, _B)],
                    rbuf.at[pl.ds(b * _B, _B)], semr))
            for b in range(G):
                j = g * G + b
                ld[b].wait()
                row = idxv.at[wid * nb + j] if full_idx else idxv.at[j]
                pltpu.sync_copy(rbuf.at[pl.ds(b * _B, _B)],
                                shared.at[row], add=True)
                if with_counts:
                    pltpu.sync_copy(obuf, shared_cnt.at[row], add=True)
            return carry

        lax.fori_loop(0, nb // G, group, 0)
        plsc.subcore_barrier()

        @pl.when(c == 0)
        def _():
            pltpu.sync_copy(shared.at[pl.ds(s * rpt, rpt)], stage)
            pltpu.sync_copy(stage, s0_hbm.at[pl.ds(s * rpt, rpt)])
            if with_counts:
                pltpu.sync_copy(shared_cnt.at[pl.ds(s * rpt, rpt)], stage)
                pltpu.sync_copy(stage, c0_hbm.at[pl.ds(s * rpt, rpt)])

        @pl.when(c == 1)
        def _():
            pltpu.sync_copy(shared.at[pl.ds(s * rpt, rpt)], stage)
            pltpu.sync_copy(stage, s1_hbm.at[pl.ds(s * rpt, rpt)])
            if with_counts:
                pltpu.sync_copy(shared_cnt.at[pl.ds(s * rpt, rpt)], stage)
                pltpu.sync_copy(stage, c1_hbm.at[pl.ds(s * rpt, rpt)])

    idx_rows = (R // _B) if full_idx else nb
    scratch = [
        pltpu.VMEM((idx_rows, _B), jnp.int32),
        pltpu.VMEM((max(G * _B, rpt), D), _f32),
    ]
    if with_counts:
        scratch.append(pltpu.VMEM((_B, D), _f32))
    scratch.append(pltpu.VMEM_SHARED((T, D), _f32))
    if with_counts:
        scratch.append(pltpu.VMEM_SHARED((T, D), _f32))
    scratch.append(pltpu.SemaphoreType.DMA)

    return pl.kernel(
        body,
        out_type=[jax.ShapeDtypeStruct((T, D), _f32)] * n_out,
        mesh=_sc_mesh(),
        compiler_params=pltpu.CompilerParams(use_tc_tiling_on_sc=False),
        scratch_types=scratch,
    )


def kernel(x, edge_index, edge_attr, batch, W_emb, b_emb, W_full, b_full,
           g1, be1, g2, be2, W_fc, b_fc, W_out, b_out):
    N = x.shape[0]
    E = edge_index.shape[1]
    d = W_emb.shape[1]
    n_conv = W_full.shape[0]

    grain = _NW * _B
    egrain = grain * 8
    Epad = ((E + egrain - 1) // egrain) * egrain
    Tn = ((N + _B - 1) // _B) * _B
    if Tn == N:
        Tn = N + _B

    src = edge_index[0]
    dst = edge_index[1]
    epad = Epad - E
    srcg = jnp.concatenate([src, jnp.zeros((epad,), jnp.int32)])
    dstg = jnp.concatenate([dst, jnp.zeros((epad,), jnp.int32)])
    dsts = jnp.concatenate([dst, jnp.full((epad,), N, jnp.int32)])
    srcg = srcg.reshape(Epad // _B, _B)
    dstg = dstg.reshape(Epad // _B, _B)
    dsts = dsts.reshape(Epad // _B, _B)

    h = _embed(x, W_emb, b_emb)

    S = 4
    Eh = Epad // S
    rh = Eh // _B
    realE = [max(0, min(E - i * Eh, Eh)) for i in range(S)]
    gather = _make_gather(N, Eh, d)
    scat_msg = _make_scatter(Eh, Tn, d, with_counts=False, full_idx=False)
    zeros_T = jnp.zeros((Tn, d), _f32)

    nbr = edge_attr.shape[1]
    ea_pad = jnp.concatenate([edge_attr, jnp.zeros((epad, nbr), _f32)])
    ea2_h = [ea_pad[i * Eh:(i + 1) * Eh].reshape(Eh // 2, 2 * nbr)
             for i in range(S)]
    dstg_h = [dstg[i * rh:(i + 1) * rh] for i in range(S)]
    srcg_h = [srcg[i * rh:(i + 1) * rh] for i in range(S)]
    dsts_h = [dsts[i * rh:(i + 1) * rh] for i in range(S)]

    def blockdiag(w):
        z = jnp.zeros_like(w)
        return jnp.concatenate(
            [jnp.concatenate([w, z], axis=1),
             jnp.concatenate([z, w], axis=1)], axis=0)

    for l in range(n_conv):
        wi2 = blockdiag(W_full[l, :d])
        wj2 = blockdiag(W_full[l, d:2 * d])
        we2 = blockdiag(W_full[l, 2 * d:])
        bias2 = jnp.concatenate([b_full[l], b_full[l]]).reshape(1, -1)
        xs, sts = [], []
        for i in range(S):
            xi, xj = gather(h, dstg_h[i], srcg_h[i])
            xs.append((xi.reshape(Eh // 2, 2 * d),
                       xj.reshape(Eh // 2, 2 * d)))
            sts.append(_edge_stats(realE[i], d, nbr, xs[i][0], xs[i][1],
                                   ea2_h[i], wi2, wj2, we2, bias2))
        st_all = sts[0]
        for stx in sts[1:]:
            st_all = st_all + stx
        parts = []
        for i in range(S):
            msg2 = _edge_msg(realE[i], E, d, nbr, xs[i][0], xs[i][1],
                             ea2_h[i], wi2, wj2, we2, bias2,
                             st_all, g1[l], be1[l])
            p0, p1 = scat_msg(msg2.reshape(Eh, d), dsts_h[i], zeros_T)
            parts += [p0, p1]
        h = _update(parts, h, g2[l], be2[l])

    NP = ((N + grain - 1) // grain) * grain
    Tg = ((_NGRAPH // _NS) + 8) * _NS
    h_pad = jnp.concatenate([h, jnp.zeros((NP - N, d), _f32)], axis=0)
    b_pad = jnp.concatenate(
        [batch, jnp.full((NP - N,), _NGRAPH, jnp.int32)]).reshape(
            NP // _B, _B)
    scat_pool = _make_scatter(NP, Tg, d, with_counts=True, full_idx=True)
    s0, s1, c0, c1 = scat_pool(h_pad, b_pad, jnp.zeros((Tg, d), _f32),
                               jnp.ones((_B, d), _f32))

    hdim = W_fc.shape[1]
    wout_pad = jnp.concatenate(
        [W_out, jnp.zeros((hdim, hdim - W_out.shape[1]), _f32)], axis=1)
    bout_pad = jnp.concatenate(
        [b_out, jnp.zeros((hdim - b_out.shape[0],), _f32)]).reshape(1, -1)
    out_full = _head(s0, s1, c0, c1,
                     W_fc, b_fc.reshape(1, -1), wout_pad, bout_pad)
    return out_full[:, :W_out.shape[1]]
